# baseline (speedup 1.0000x reference)
import jax
import jax.numpy as jnp
from jax.experimental import pallas as pl
from jax.experimental.pallas import tpu as pltpu

_STREAMS = 5
_BM = 80


def _gcn_kernel(x_ref, w_ref, *rest):
    adj_refs = rest[:_STREAMS]
    b_ref = rest[_STREAMS]
    out_ref = rest[_STREAMS + 1]
    support_ref = rest[_STREAMS + 2]
    i = pl.program_id(0)

    @pl.when(i == 0)
    def _():
        support_ref[...] = jnp.dot(
            x_ref[...], w_ref[...], preferred_element_type=jnp.float32
        ).astype(jnp.bfloat16)

    s = support_ref[...]
    b = b_ref[...]
    for j in range(_STREAMS):
        acc = jnp.dot(
            adj_refs[j][...].astype(jnp.bfloat16),
            s,
            preferred_element_type=jnp.float32,
        )
        z = acc + b
        out_ref[j * _BM : (j + 1) * _BM, :] = jnp.where(
            z > 0, z, jnp.exp(z) - 1.0
        )


def kernel(inputs, adj, weight, bias):
    n, in_f = inputs.shape
    out_f = weight.shape[1]
    bm = _BM
    ns = _STREAMS
    bias2 = bias.reshape(1, out_f)

    def adj_spec(j):
        return pl.BlockSpec((bm, n), lambda i, j=j: (ns * i + j, 0))

    return pl.pallas_call(
        _gcn_kernel,
        grid=(n // (ns * bm),),
        in_specs=[
            pl.BlockSpec((n, in_f), lambda i: (0, 0)),
            pl.BlockSpec((in_f, out_f), lambda i: (0, 0)),
            *[adj_spec(j) for j in range(ns)],
            pl.BlockSpec((1, out_f), lambda i: (0, 0)),
        ],
        out_specs=pl.BlockSpec((ns * bm, out_f), lambda i: (i, 0)),
        out_shape=jax.ShapeDtypeStruct((n, out_f), jnp.float32),
        scratch_shapes=[pltpu.VMEM((n, out_f), jnp.bfloat16)],
    )(inputs, weight, *([adj] * ns), bias2)


# speedup vs baseline: 1.0125x; 1.0125x over previous
import jax
import jax.numpy as jnp
from jax.experimental import pallas as pl
from jax.experimental.pallas import tpu as pltpu

_BM = 200
_DEPTH = 4


def _gcn_kernel(x_ref, w_ref, adj_ref, b_ref, out_ref, support_ref, buf_ref, sems):
    i = pl.program_id(0)
    nsteps = pl.num_programs(0)

    def issue(block, slot):
        pltpu.make_async_copy(
            adj_ref.at[pl.ds(block * _BM, _BM), :],
            buf_ref.at[slot],
            sems.at[slot],
        ).start()

    @pl.when(i == 0)
    def _():
        support_ref[...] = jnp.dot(
            x_ref[...], w_ref[...], preferred_element_type=jnp.float32
        ).astype(jnp.bfloat16)
        for k in range(_DEPTH - 1):
            issue(k, k)

    nb = i + _DEPTH - 1

    @pl.when(nb < nsteps)
    def _():
        issue(nb, jax.lax.rem(nb, _DEPTH))

    slot = jax.lax.rem(i, _DEPTH)
    pltpu.make_async_copy(
        adj_ref.at[pl.ds(i * _BM, _BM), :],
        buf_ref.at[slot],
        sems.at[slot],
    ).wait()

    acc = jnp.dot(
        buf_ref[slot].astype(jnp.bfloat16),
        support_ref[...],
        preferred_element_type=jnp.float32,
    )
    z = acc + b_ref[...]
    out_ref[...] = jnp.where(z > 0, z, jnp.exp(z) - 1.0)


def kernel(inputs, adj, weight, bias):
    n, in_f = inputs.shape
    out_f = weight.shape[1]
    bm = _BM
    bias2 = bias.reshape(1, out_f)
    return pl.pallas_call(
        _gcn_kernel,
        grid=(n // bm,),
        in_specs=[
            pl.BlockSpec((n, in_f), lambda i: (0, 0)),
            pl.BlockSpec((in_f, out_f), lambda i: (0, 0)),
            pl.BlockSpec(memory_space=pltpu.MemorySpace.HBM),
            pl.BlockSpec((1, out_f), lambda i: (0, 0)),
        ],
        out_specs=pl.BlockSpec((bm, out_f), lambda i: (i, 0)),
        out_shape=jax.ShapeDtypeStruct((n, out_f), jnp.float32),
        scratch_shapes=[
            pltpu.VMEM((n, out_f), jnp.bfloat16),
            pltpu.VMEM((_DEPTH, bm, n), jnp.float32),
            pltpu.SemaphoreType.DMA((_DEPTH,)),
        ],
    )(inputs, weight, adj, bias2)


# speedup vs baseline: 1.0281x; 1.0154x over previous
import jax
import jax.numpy as jnp
from jax.experimental import pallas as pl
from jax.experimental.pallas import tpu as pltpu

_BM = 400


def _gcn_kernel(x_ref, w_ref, adj_ref, b_ref, out_ref, support_ref):
    i = pl.program_id(0)

    @pl.when(i == 0)
    def _():
        support_ref[...] = jnp.dot(
            x_ref[...], w_ref[...], preferred_element_type=jnp.float32
        ).astype(jnp.bfloat16)

    acc = jnp.dot(
        adj_ref[...].astype(jnp.bfloat16),
        support_ref[...],
        preferred_element_type=jnp.float32,
    )
    z = acc + b_ref[...]
    out_ref[...] = jnp.where(z > 0, z, jnp.exp(z) - 1.0)


def kernel(inputs, adj, weight, bias):
    n, in_f = inputs.shape
    out_f = weight.shape[1]
    bm = _BM
    bias2 = bias.reshape(1, out_f)
    return pl.pallas_call(
        _gcn_kernel,
        grid=(n // bm,),
        in_specs=[
            pl.BlockSpec((n, in_f), lambda i: (0, 0)),
            pl.BlockSpec((in_f, out_f), lambda i: (0, 0)),
            pl.BlockSpec((bm, n), lambda i: (i, 0)),
            pl.BlockSpec((1, out_f), lambda i: (0, 0)),
        ],
        out_specs=pl.BlockSpec((bm, out_f), lambda i: (i, 0)),
        out_shape=jax.ShapeDtypeStruct((n, out_f), jnp.float32),
        scratch_shapes=[pltpu.VMEM((n, out_f), jnp.bfloat16)],
    )(inputs, weight, adj, bias2)
